# Initial kernel scaffold; baseline (speedup 1.0000x reference)
import jax
import jax.numpy as jnp
from jax.experimental import pallas as pl


def kernel(X_target, X_context1, X_context2, W1, b1, W2, b2):
    raise NotImplementedError("write your pallas kernel here")


# speedup vs baseline: 3.6382x; 3.6382x over previous
import jax
import jax.numpy as jnp
from jax.experimental import pallas as pl

THRESH = 0.1


def _hg_batch_kernel(xt_ref, xc_ref, xself_ref, xnbr_ref,
                     w1_ref, b1_ref, w2_ref, b2_ref,
                     outu_ref, outsa_ref, outsb_ref):
    xt = xt_ref[0]
    xc = xc_ref[0]
    w1 = w1_ref[...]
    b1 = b1_ref[...]
    w2 = w2_ref[...]
    b2 = b2_ref[...]

    tn = xt / jnp.maximum(jnp.sqrt(jnp.sum(xt * xt, axis=1, keepdims=True)), 1e-8)
    cn = xc / jnp.maximum(jnp.sqrt(jnp.sum(xc * xc, axis=1, keepdims=True)), 1e-8)
    nt_dims = (((1,), (1,)), ((), ()))
    sim = jax.lax.dot_general(tn, cn, nt_dims,
                              preferred_element_type=jnp.float32)
    sim_t = jax.lax.dot_general(cn, tn, nt_dims,
                                preferred_element_type=jnp.float32)
    s_mask = (sim > THRESH).astype(jnp.float32)
    s_mask_t = (sim_t > THRESH).astype(jnp.float32)

    d_e = 1.0 + jnp.sum(s_mask, axis=1, keepdims=True)

    t_self = jnp.dot(xself_ref[0], w1, preferred_element_type=jnp.float32) + b1
    t_nbr = jnp.dot(xnbr_ref[0], w1, preferred_element_type=jnp.float32) + b1

    x_edge = (t_self + jnp.dot(s_mask, t_nbr,
                               preferred_element_type=jnp.float32)) / d_e
    u = jnp.dot(x_edge, w2, preferred_element_type=jnp.float32) + b2
    outu_ref[0] = u

    stu = jnp.dot(s_mask_t, u, preferred_element_type=jnp.float32)
    d_v = jnp.maximum(jnp.sum(s_mask_t, axis=1, keepdims=True), 1.0)
    outs = stu / d_v
    nt = outu_ref.shape[1]
    outsa_ref[0] = outs[:nt]
    outsb_ref[0] = outs[nt:]


def kernel(X_target, X_context1, X_context2, W1, b1, W2, b2):
    B, C, Hh, Ww = X_target.shape
    N = Hh * Ww
    to_rows = lambda a: jnp.transpose(a, (0, 2, 3, 1)).reshape(B, N, C)
    Xt = to_rows(X_target)
    Xc1 = to_rows(X_context1)
    Xc2 = to_rows(X_context2)
    Xc = jnp.concatenate([Xc1, Xc2], axis=1)

    Xself = jnp.stack([Xt[0], Xc2[0]])
    Xnbr = jnp.stack([jnp.concatenate([Xt[1], Xc1[0]], axis=0),
                      jnp.concatenate([Xc1[1], Xc2[1]], axis=0)])

    b1r = b1.reshape(1, C)
    b2r = b2.reshape(1, C)

    outu, outsa, outsb = pl.pallas_call(
        _hg_batch_kernel,
        grid=(B,),
        in_specs=[
            pl.BlockSpec((1, N, C), lambda i: (i, 0, 0)),
            pl.BlockSpec((1, 2 * N, C), lambda i: (i, 0, 0)),
            pl.BlockSpec((1, N, C), lambda i: (i, 0, 0)),
            pl.BlockSpec((1, 2 * N, C), lambda i: (i, 0, 0)),
            pl.BlockSpec((C, C), lambda i: (0, 0)),
            pl.BlockSpec((1, C), lambda i: (0, 0)),
            pl.BlockSpec((C, C), lambda i: (0, 0)),
            pl.BlockSpec((1, C), lambda i: (0, 0)),
        ],
        out_specs=[
            pl.BlockSpec((1, N, C), lambda i: (i, 0, 0)),
            pl.BlockSpec((1, N, C), lambda i: (i, 0, 0)),
            pl.BlockSpec((1, N, C), lambda i: (i, 0, 0)),
        ],
        out_shape=[
            jax.ShapeDtypeStruct((B, N, C), jnp.float32),
            jax.ShapeDtypeStruct((B, N, C), jnp.float32),
            jax.ShapeDtypeStruct((B, N, C), jnp.float32),
        ],
    )(Xt, Xc, Xself, Xnbr, W1, b1r, W2, b2r)

    Yt = jnp.stack([outu[0], outsa[0]])
    Yc1 = jnp.stack([outsb[0], outsa[1]])
    Yc2 = jnp.stack([outu[1], outsb[1]])
    to_nchw = lambda a: jnp.transpose(a.reshape(B, Hh, Ww, C), (0, 3, 1, 2))
    return (to_nchw(Yt), to_nchw(Yc1), to_nchw(Yc2))


# speedup vs baseline: 4.6881x; 1.2886x over previous
import jax
import jax.numpy as jnp
from jax.experimental import pallas as pl

THRESH = 0.1

_TN = (((0,), (0,)), ((), ()))
_NN = (((1,), (0,)), ((), ()))


def _dot(a, b, dims):
    return jax.lax.dot_general(a, b, dims, preferred_element_type=jnp.float32)


def _normalize_cols(x):
    n = jnp.maximum(jnp.sqrt(jnp.sum(x * x, axis=0, keepdims=True)), 1e-8)
    return x / n


def _hg_kernel(xt_ref, xc1_ref, xc2_ref, w1_ref, b1_ref, w2_ref, b2_ref,
               yt_ref, yc1_ref, yc2_ref):
    w1 = w1_ref[...]
    b1 = b1_ref[...]
    w2 = w2_ref[...]
    b2 = b2_ref[...]

    selfs = (xt_ref[0], xc2_ref[0])
    nbrs = ((xt_ref[1], xc1_ref[0]), (xc1_ref[1], xc2_ref[1]))

    for i in range(2):
        tn = _normalize_cols(xt_ref[i])
        ca = _normalize_cols(xc1_ref[i])
        cb = _normalize_cols(xc2_ref[i])

        m_a = (_dot(tn, ca, _TN) > THRESH).astype(jnp.float32)
        m_b = (_dot(tn, cb, _TN) > THRESH).astype(jnp.float32)
        mt_a = (_dot(ca, tn, _TN) > THRESH).astype(jnp.float32)
        mt_b = (_dot(cb, tn, _TN) > THRESH).astype(jnp.float32)

        d_e = (1.0 + jnp.sum(mt_a, axis=0, keepdims=True)
               + jnp.sum(mt_b, axis=0, keepdims=True))

        t_self = _dot(w1, selfs[i], _TN) + b1
        t_na = _dot(w1, nbrs[i][0], _TN) + b1
        t_nb = _dot(w1, nbrs[i][1], _TN) + b1

        x_edge = (t_self + _dot(t_na, mt_a, _NN) + _dot(t_nb, mt_b, _NN)) / d_e
        u = _dot(w2, x_edge, _TN) + b2

        d_va = jnp.maximum(jnp.sum(m_a, axis=0, keepdims=True), 1.0)
        d_vb = jnp.maximum(jnp.sum(m_b, axis=0, keepdims=True), 1.0)
        s_a = _dot(u, m_a, _NN) / d_va
        s_b = _dot(u, m_b, _NN) / d_vb

        if i == 0:
            yt_ref[0] = u
            yt_ref[1] = s_a
            yc1_ref[0] = s_b
        else:
            yc2_ref[0] = u
            yc1_ref[1] = s_a
            yc2_ref[1] = s_b


def kernel(X_target, X_context1, X_context2, W1, b1, W2, b2):
    B, C, Hh, Ww = X_target.shape
    N = Hh * Ww
    xt = X_target.reshape(B, C, N)
    xc1 = X_context1.reshape(B, C, N)
    xc2 = X_context2.reshape(B, C, N)
    b1c = b1.reshape(C, 1)
    b2c = b2.reshape(C, 1)

    shp = jax.ShapeDtypeStruct((B, C, N), jnp.float32)
    yt, yc1, yc2 = pl.pallas_call(
        _hg_kernel,
        out_shape=[shp, shp, shp],
    )(xt, xc1, xc2, W1, b1c, W2, b2c)

    rs = lambda a: a.reshape(B, C, Hh, Ww)
    return (rs(yt), rs(yc1), rs(yc2))
